# baseline (device time: 19625 ns/iter reference)
import jax
import jax.numpy as jnp
from jax import lax
from jax.experimental import pallas as pl
from jax.experimental.pallas import tpu as pltpu

Y_DEV = 4
NCHUNKS = 8


def kernel(x, W, labels):
    T, D = x.shape
    _, V = W.shape
    C = V // NCHUNKS
    labels2 = labels.reshape(T, 1)

    def body(x_ref, w_ref, lab_ref, out_ref,
             xb_ref, stats_ref, recv_ref, send_sems, recv_sems):
        my_x = lax.axis_index("x")
        my_y = lax.axis_index("y")
        my_z = lax.axis_index("z")
        pi = pl.program_id(0)

        @pl.when(pi == 0)
        def _():
            barrier_sem = pltpu.get_barrier_semaphore()
            for dy in (1, 2, 3):
                pl.semaphore_signal(
                    barrier_sem, inc=1,
                    device_id=(my_x, (my_y + dy) % Y_DEV, my_z),
                    device_id_type=pl.DeviceIdType.MESH,
                )
            xb_ref[...] = x_ref[...].astype(jnp.bfloat16)
            stats_ref[...] = jnp.zeros_like(stats_ref)

        wb = w_ref[...].astype(jnp.bfloat16)
        logits = jnp.dot(xb_ref[...], wb,
                         preferred_element_type=jnp.float32)
        s = jnp.sum(jnp.exp(logits), axis=1, keepdims=True)
        cols = lax.broadcasted_iota(jnp.int32, (T, C), 1)
        lab = lab_ref[...] - (my_y * V + pi * C)
        lsel = jnp.sum(jnp.where(cols == lab, logits, 0.0), axis=1,
                       keepdims=True)
        stats_ref[:, 0:1] += s
        stats_ref[:, 1:2] += lsel

        @pl.when(pi == NCHUNKS - 1)
        def _():
            pl.semaphore_wait(pltpu.get_barrier_semaphore(), 3)
            rdmas = []
            for dy in (1, 2, 3):
                rdma = pltpu.make_async_remote_copy(
                    src_ref=stats_ref,
                    dst_ref=recv_ref.at[dy - 1],
                    send_sem=send_sems.at[dy - 1],
                    recv_sem=recv_sems.at[dy - 1],
                    device_id=(my_x, (my_y + dy) % Y_DEV, my_z),
                    device_id_type=pl.DeviceIdType.MESH,
                )
                rdma.start()
                rdmas.append(rdma)
            for rdma in rdmas:
                rdma.wait()

            s_g = stats_ref[:, 0:1]
            l_g = stats_ref[:, 1:2]
            for j in range(3):
                s_g = s_g + recv_ref[j, :, 0:1]
                l_g = l_g + recv_ref[j, :, 1:2]
            nll = jnp.log(s_g) - l_g
            out_ref[...] = nll[:, 0]

    return pl.pallas_call(
        body,
        grid=(NCHUNKS,),
        out_shape=jax.ShapeDtypeStruct((T,), jnp.float32),
        in_specs=[
            pl.BlockSpec((T, D), lambda i: (0, 0), memory_space=pltpu.VMEM),
            pl.BlockSpec((D, C), lambda i: (0, i), memory_space=pltpu.VMEM),
            pl.BlockSpec((T, 1), lambda i: (0, 0), memory_space=pltpu.VMEM),
        ],
        out_specs=pl.BlockSpec((T,), lambda i: (0,), memory_space=pltpu.VMEM),
        scratch_shapes=[
            pltpu.VMEM((T, D), jnp.bfloat16),
            pltpu.VMEM((T, 2), jnp.float32),
            pltpu.VMEM((3, T, 2), jnp.float32),
            pltpu.SemaphoreType.DMA((3,)),
            pltpu.SemaphoreType.DMA((3,)),
        ],
        compiler_params=pltpu.CompilerParams(
            collective_id=0,
            dimension_semantics=("arbitrary",),
        ),
    )(x, W, labels2)


# device time: 13179 ns/iter; 1.4891x vs baseline; 1.4891x over previous
import jax
import jax.numpy as jnp
from jax import lax
from jax.experimental import pallas as pl
from jax.experimental.pallas import tpu as pltpu

Y_DEV = 4
NCHUNKS = 1


def kernel(x, W, labels):
    T, D = x.shape
    _, V = W.shape
    C = V // NCHUNKS
    labels2 = labels.reshape(T, 1)

    def body(x_ref, w_ref, lab_ref, out_ref,
             xb_ref, stats_ref, recv_ref, send_sems, recv_sems):
        my_x = lax.axis_index("x")
        my_y = lax.axis_index("y")
        my_z = lax.axis_index("z")
        pi = pl.program_id(0)

        @pl.when(pi == 0)
        def _():
            barrier_sem = pltpu.get_barrier_semaphore()
            for dy in (1, 2, 3):
                pl.semaphore_signal(
                    barrier_sem, inc=1,
                    device_id=(my_x, (my_y + dy) % Y_DEV, my_z),
                    device_id_type=pl.DeviceIdType.MESH,
                )
            xb_ref[...] = x_ref[...].astype(jnp.bfloat16)
            stats_ref[...] = jnp.zeros_like(stats_ref)

        wb = w_ref[...].astype(jnp.bfloat16)
        logits = jnp.dot(xb_ref[...], wb,
                         preferred_element_type=jnp.float32)
        s = jnp.sum(jnp.exp(logits), axis=1)
        cols = lax.broadcasted_iota(jnp.int32, (T, C), 1)
        lab = lab_ref[...] - (my_y * V + pi * C)
        lsel = jnp.sum(jnp.where(cols == lab, logits, 0.0), axis=1)
        stats_ref[0:2, :] += s.reshape(2, 128)
        stats_ref[2:4, :] += lsel.reshape(2, 128)

        @pl.when(pi == NCHUNKS - 1)
        def _():
            pl.semaphore_wait(pltpu.get_barrier_semaphore(), 3)
            rdmas = []
            for dy in (1, 2, 3):
                rdma = pltpu.make_async_remote_copy(
                    src_ref=stats_ref,
                    dst_ref=recv_ref.at[dy - 1],
                    send_sem=send_sems.at[dy - 1],
                    recv_sem=recv_sems.at[dy - 1],
                    device_id=(my_x, (my_y + dy) % Y_DEV, my_z),
                    device_id_type=pl.DeviceIdType.MESH,
                )
                rdma.start()
                rdmas.append(rdma)
            for rdma in rdmas:
                rdma.wait()

            tot = stats_ref[...]
            for j in range(3):
                tot = tot + recv_ref[j]
            nll = jnp.log(tot[0:2, :]) - tot[2:4, :]
            out_ref[...] = nll.reshape(T)

    return pl.pallas_call(
        body,
        grid=(NCHUNKS,),
        out_shape=jax.ShapeDtypeStruct((T,), jnp.float32),
        in_specs=[
            pl.BlockSpec((T, D), lambda i: (0, 0), memory_space=pltpu.VMEM),
            pl.BlockSpec((D, C), lambda i: (0, i), memory_space=pltpu.VMEM),
            pl.BlockSpec((T, 1), lambda i: (0, 0), memory_space=pltpu.VMEM),
        ],
        out_specs=pl.BlockSpec((T,), lambda i: (0,), memory_space=pltpu.VMEM),
        scratch_shapes=[
            pltpu.VMEM((T, D), jnp.bfloat16),
            pltpu.VMEM((4, 128), jnp.float32),
            pltpu.VMEM((3, 4, 128), jnp.float32),
            pltpu.SemaphoreType.DMA((3,)),
            pltpu.SemaphoreType.DMA((3,)),
        ],
        compiler_params=pltpu.CompilerParams(
            collective_id=0,
            dimension_semantics=("arbitrary",),
        ),
    )(x, W, labels2)
